# Initial kernel scaffold; baseline (speedup 1.0000x reference)
import jax
import jax.numpy as jnp
from jax.experimental import pallas as pl


def kernel(x, edge_index, edge_type, nest_tensor, food_tensor, W_rel, W_root, b_rgcn, W_fc, b_fc):
    raise NotImplementedError("write your pallas kernel here")


# speedup vs baseline: 16.8353x; 16.8353x over previous
import jax
import jax.numpy as jnp
from jax import lax
from jax.experimental import pallas as pl
from jax.experimental.pallas import tpu as pltpu
from jax.experimental.pallas import tpu_sc as plsc

N = 10000
E = 320000
D = 128
R = 4
H = 256
B = 1024

NC = 2
NS = 16
LANES = 16

NODES = 2 * B
SENTINEL = NODES
SLOT_PAD = 10240
ACC_ROWS = 8320
ROWS_PER_SUB = ACC_ROWS // NS
SLOT_PER_SUB = SLOT_PAD // NS
CHUNK = 128
CHUNKS_PER_SUB = 79
EDGES_PER_SUB = CHUNKS_PER_SUB * CHUNK
E_PAD = NC * NS * EDGES_PER_SUB
NODES_PER_SUB = NODES // NS
XN_PER_W = NODES // (NC * NS)


def _sc_body(x_hbm, src_hbm, dst_hbm, et_hbm, nodes_hbm,
             acc2_hbm, cnt2_hbm, xn_hbm,
             slot_sh, acc_sh, cnt_sh,
             zrows, zline, sbuf, nv, vals,
             sv, dv, ev, sd, pv, ones, xrows,
             g, gp, cv, nv2, xnrows, sem):
    c = lax.axis_index("c")
    s = lax.axis_index("s")
    wid = s * NC + c

    zero16 = jnp.zeros((LANES,), jnp.float32)
    iota16 = lax.iota(jnp.int32, LANES)

    sent16 = jnp.full((LANES,), SENTINEL, jnp.int32)
    one16 = jnp.ones((LANES,), jnp.float32)

    def zrow_body(i, carry):
        for k in range(D // LANES):
            zrows[i, pl.ds(k * LANES, LANES)] = zero16
        return carry

    lax.fori_loop(0, CHUNK, zrow_body, 0)
    for k in range(SLOT_PER_SUB // LANES):
        zline[pl.ds(k * LANES, LANES)] = zero16
        sbuf[pl.ds(k * LANES, LANES)] = sent16
    for k in range(CHUNK // LANES):
        ones[pl.ds(k * LANES, LANES)] = one16

    base_r = s * ROWS_PER_SUB
    for j in range(4):
        pltpu.sync_copy(zrows, acc_sh.at[pl.ds(base_r + j * CHUNK, CHUNK)])
    pltpu.sync_copy(zrows.at[pl.ds(0, 8)], acc_sh.at[pl.ds(base_r + 512, 8)])
    pltpu.sync_copy(zline.at[pl.ds(0, ROWS_PER_SUB)],
                    cnt_sh.at[pl.ds(base_r, ROWS_PER_SUB)])
    pltpu.sync_copy(sbuf, slot_sh.at[pl.ds(s * SLOT_PER_SUB, SLOT_PER_SUB)])

    plsc.subcore_barrier()

    nbase = s * NODES_PER_SUB
    pltpu.sync_copy(nodes_hbm.at[pl.ds(nbase, NODES_PER_SUB)], nv)
    for k in range(8):
        vals[pl.ds(k * LANES, LANES)] = nbase + k * LANES + iota16
    pltpu.sync_copy(vals, slot_sh.at[nv])

    plsc.subcore_barrier()

    ebase = wid * EDGES_PER_SUB

    def chunk_body(t, carry):
        b0 = ebase + t * CHUNK
        pltpu.sync_copy(src_hbm.at[pl.ds(b0, CHUNK)], sv)
        pltpu.sync_copy(dst_hbm.at[pl.ds(b0, CHUNK)], dv)
        pltpu.sync_copy(et_hbm.at[pl.ds(b0, CHUNK)], ev)
        pltpu.sync_copy(slot_sh.at[dv], sd)
        for k in range(8):
            sl = pl.ds(k * LANES, LANES)
            pv[sl] = sd[sl] * R + ev[sl]
        pltpu.async_copy(x_hbm.at[sv], xrows, sem).wait()
        pltpu.sync_copy(xrows, acc_sh.at[pv], add=True)
        pltpu.sync_copy(ones, cnt_sh.at[pv], add=True)
        return carry

    lax.fori_loop(0, CHUNKS_PER_SUB, chunk_body, 0)

    plsc.subcore_barrier()

    pltpu.sync_copy(slot_sh.at[nv], g)
    for r in range(R):
        for k in range(8):
            sl = pl.ds(k * LANES, LANES)
            gp[sl] = g[sl] * R + r
        pltpu.sync_copy(acc_sh.at[gp], xrows)
        fbase = (c * R + r) * NODES + nbase
        pltpu.sync_copy(xrows, acc2_hbm.at[pl.ds(fbase, NODES_PER_SUB)])
        pltpu.sync_copy(cnt_sh.at[gp], cv)
        pltpu.sync_copy(cv, cnt2_hbm.at[pl.ds(fbase, NODES_PER_SUB)])

    xb = wid * XN_PER_W
    pltpu.sync_copy(nodes_hbm.at[pl.ds(xb, XN_PER_W)], nv2)
    pltpu.async_copy(x_hbm.at[nv2], xnrows, sem).wait()
    pltpu.sync_copy(xnrows, xn_hbm.at[pl.ds(xb, XN_PER_W)])


_SC_SCRATCH = [
        pltpu.VMEM_SHARED((SLOT_PAD,), jnp.int32),
        pltpu.VMEM_SHARED((ACC_ROWS, D), jnp.float32),
        pltpu.VMEM_SHARED((ACC_ROWS,), jnp.float32),
        pltpu.VMEM((CHUNK, D), jnp.float32),
        pltpu.VMEM((SLOT_PER_SUB,), jnp.float32),
        pltpu.VMEM((SLOT_PER_SUB,), jnp.int32),
        pltpu.VMEM((NODES_PER_SUB,), jnp.int32),
        pltpu.VMEM((NODES_PER_SUB,), jnp.int32),
        pltpu.VMEM((CHUNK,), jnp.int32),
        pltpu.VMEM((CHUNK,), jnp.int32),
        pltpu.VMEM((CHUNK,), jnp.int32),
        pltpu.VMEM((CHUNK,), jnp.int32),
        pltpu.VMEM((CHUNK,), jnp.int32),
        pltpu.VMEM((CHUNK,), jnp.float32),
        pltpu.VMEM((CHUNK, D), jnp.float32),
        pltpu.VMEM((CHUNK,), jnp.int32),
        pltpu.VMEM((CHUNK,), jnp.int32),
        pltpu.VMEM((CHUNK,), jnp.float32),
        pltpu.VMEM((XN_PER_W,), jnp.int32),
        pltpu.VMEM((XN_PER_W, D), jnp.float32),
        pltpu.SemaphoreType.DMA,
]

_SC_OUT = (
    jax.ShapeDtypeStruct((NC * R * NODES, D), jnp.float32),
    jax.ShapeDtypeStruct((NC * R * NODES,), jnp.float32),
    jax.ShapeDtypeStruct((NODES, D), jnp.float32),
)

_sc_call_cached = None


def _sc_call(*args):
    global _sc_call_cached
    if _sc_call_cached is None:
        _sc_call_cached = pl.kernel(
            _sc_body,
            out_type=_SC_OUT,
            mesh=plsc.VectorSubcoreMesh(core_axis_name="c",
                                        subcore_axis_name="s",
                                        num_cores=NC, num_subcores=NS),
            scratch_types=_SC_SCRATCH,
        )
    return _sc_call_cached(*args)


def _tc_body(acc2_ref, cnt2_ref, xn_ref, wrel_ref, wroot_ref, brg_ref,
             wfc_ref, bfc_ref, out_ref):
    agg = jnp.zeros((NODES, D), jnp.float32)
    for r in range(R):
        acc_r = acc2_ref[0, r] + acc2_ref[1, r]
        cnt_r = cnt2_ref[0, r] + cnt2_ref[1, r]
        norm = 1.0 / jnp.maximum(cnt_r, 1.0)
        agg = agg + jnp.dot(acc_r * norm, wrel_ref[r],
                            preferred_element_type=jnp.float32)
    h = agg + jnp.dot(xn_ref[...], wroot_ref[...],
                      preferred_element_type=jnp.float32) + brg_ref[...]
    h = jnp.maximum(h, 0.0)
    comb = jnp.concatenate([h[:B], h[B:]], axis=1)
    out = jnp.dot(comb, wfc_ref[...],
                  preferred_element_type=jnp.float32) + bfc_ref[...]
    out_ref[...] = jnp.maximum(out, 0.0)


def _tc_call(acc2, cnt2, xn, W_rel, W_root, brg, W_fc, bfc):
    return pl.pallas_call(
        _tc_body,
        out_shape=jax.ShapeDtypeStruct((B, H), jnp.float32),
    )(acc2, cnt2, xn, W_rel, W_root, brg, W_fc, bfc)


def kernel(x, edge_index, edge_type, nest_tensor, food_tensor,
           W_rel, W_root, b_rgcn, W_fc, b_fc):
    src = edge_index[0].astype(jnp.int32)
    dst = edge_index[1].astype(jnp.int32)
    et = edge_type.astype(jnp.int32)
    nodes = jnp.concatenate([nest_tensor, food_tensor]).astype(jnp.int32)
    pad = E_PAD - E
    src = jnp.concatenate([src, jnp.zeros((pad,), jnp.int32)])
    dst = jnp.concatenate([dst, jnp.full((pad,), N, jnp.int32)])
    et = jnp.concatenate([et, jnp.zeros((pad,), jnp.int32)])

    acc2, cnt2, xn = _sc_call(x, src, dst, et, nodes)
    acc2 = acc2.reshape(NC, R, NODES, D)
    cnt2 = cnt2.reshape(NC, R, NODES, 1)
    return _tc_call(acc2, cnt2, xn, W_rel, W_root,
                    b_rgcn.reshape(1, D), W_fc, b_fc.reshape(1, H))


# speedup vs baseline: 17.7610x; 1.0550x over previous
import jax
import jax.numpy as jnp
from jax import lax
from jax.experimental import pallas as pl
from jax.experimental.pallas import tpu as pltpu
from jax.experimental.pallas import tpu_sc as plsc

N = 10000
E = 320000
D = 128
R = 4
H = 256
B = 1024

NC = 2
NS = 16
LANES = 16

NODES = 2 * B
SENTINEL = NODES
SLOT_PAD = 10240
ACC_ROWS = 8320
ROWS_PER_SUB = ACC_ROWS // NS
SLOT_PER_SUB = SLOT_PAD // NS
CHUNK = 128
CHUNKS_PER_SUB = 79
EDGES_PER_SUB = CHUNKS_PER_SUB * CHUNK
E_PAD = NC * NS * EDGES_PER_SUB
NODES_PER_SUB = NODES // NS
XN_PER_W = NODES // (NC * NS)
LIST = EDGES_PER_SUB + 2 * CHUNK
TRASH = LIST
LIST_CAP = LIST + LANES
DUMMY_PAIR = SENTINEL * R
PAD_PACK = DUMMY_PAIR << 16


def _sc_body(x_hbm, epk_hbm, dst_hbm, nodes_hbm,
             acc2_hbm, cnt2_hbm, xn_hbm,
             slot_sh, acc_sh, cnt_sh, cl_sh,
             epk_all, dvCA, dvCB, sdA, sdB, pkbuf, posv,
             pkC, svcA, svcB, pvcA, pvcB,
             xrowsA, xrowsB, padidx, padpk,
             zline, sbuf, ones, nv, vals, g, gp, cv, nv2,
             semA, semB, semDA, semDB):
    c = lax.axis_index("c")
    s = lax.axis_index("s")
    wid = s * NC + c
    list_base = s * LIST_CAP

    zero16 = jnp.zeros((LANES,), jnp.float32)
    iota16 = lax.iota(jnp.int32, LANES)
    sent16 = jnp.full((LANES,), SENTINEL, jnp.int32)
    one16 = jnp.ones((LANES,), jnp.float32)

    ebase = wid * EDGES_PER_SUB
    pltpu.async_copy(epk_hbm.at[pl.ds(ebase, EDGES_PER_SUB)], epk_all, semDA)

    def zrow_body(i, carry):
        for k in range(D // LANES):
            xrowsA[i, pl.ds(k * LANES, LANES)] = zero16
        return carry

    lax.fori_loop(0, CHUNK, zrow_body, 0)
    for k in range(SLOT_PER_SUB // LANES):
        zline[pl.ds(k * LANES, LANES)] = zero16
        sbuf[pl.ds(k * LANES, LANES)] = sent16
    for k in range(CHUNK // LANES):
        ones[pl.ds(k * LANES, LANES)] = one16
        padpk[pl.ds(k * LANES, LANES)] = jnp.full((LANES,), PAD_PACK,
                                                  jnp.int32)

    base_r = s * ROWS_PER_SUB
    for j in range(4):
        pltpu.sync_copy(xrowsA, acc_sh.at[pl.ds(base_r + j * CHUNK, CHUNK)])
    pltpu.sync_copy(xrowsA.at[pl.ds(0, 8)], acc_sh.at[pl.ds(base_r + 512, 8)])
    pltpu.sync_copy(zline.at[pl.ds(0, ROWS_PER_SUB)],
                    cnt_sh.at[pl.ds(base_r, ROWS_PER_SUB)])
    pltpu.sync_copy(sbuf, slot_sh.at[pl.ds(s * SLOT_PER_SUB, SLOT_PER_SUB)])

    plsc.subcore_barrier()

    nbase = s * NODES_PER_SUB
    pltpu.sync_copy(nodes_hbm.at[pl.ds(nbase, NODES_PER_SUB)], nv)
    for k in range(NODES_PER_SUB // LANES):
        vals[pl.ds(k * LANES, LANES)] = nbase + k * LANES + iota16
    pltpu.sync_copy(vals, slot_sh.at[nv])

    plsc.subcore_barrier()

    pltpu.make_async_copy(epk_hbm.at[pl.ds(0, EDGES_PER_SUB)], epk_all,
                          semDA).wait()

    def load_dv(t, buf, sem):
        pltpu.async_copy(dst_hbm.at[pl.ds(ebase + t * CHUNK, CHUNK)],
                         buf, sem)

    def wait_dv(buf, sem):
        pltpu.make_async_copy(dst_hbm.at[pl.ds(0, CHUNK)], buf, sem).wait()

    def issue_sd(dvC, buf, sem):
        pltpu.async_copy(slot_sh.at[dvC], buf, sem)

    def wait_sd(buf, sem):
        pltpu.make_async_copy(slot_sh.at[dvCA], buf, sem).wait()

    def compact_chunk(t, sdref, cur):
        for k in range(CHUNK // LANES):
            sl = pl.ds(k * LANES, LANES)
            off = t * CHUNK + k * LANES
            sd16 = sdref[sl]
            ep16 = epk_all[pl.ds(off, LANES)]
            ev16 = lax.shift_right_logical(ep16, 16)
            sv16 = ep16 & 0xFFFF
            valid = sd16 < SENTINEL
            pv16 = sd16 * R + ev16
            csum = jnp.where(valid, 1, 0).astype(jnp.int32)
            for sh in (1, 2, 4, 8):
                idx = jnp.maximum(iota16 - sh, 0)
                shifted = jnp.take(csum, idx, mode="wrap")
                csum = csum + jnp.where(iota16 >= sh, shifted, 0)
            pos16 = jnp.where(valid, list_base + cur + csum - 1,
                              list_base + TRASH + iota16)
            pkbuf[sl] = sv16 | lax.shift_left(pv16, 16)
            posv[sl] = pos16
            cur = cur + csum[LANES - 1]
        pltpu.sync_copy(pkbuf, cl_sh.at[posv])
        return cur

    last = CHUNKS_PER_SUB - 1
    load_dv(0, dvCA, semDA)
    wait_dv(dvCA, semDA)
    issue_sd(dvCA, sdA, semA)
    load_dv(1, dvCB, semDB)

    def pairA(i, cur):
        t0 = 2 * i
        wait_dv(dvCB, semDB)
        issue_sd(dvCB, sdB, semB)
        wait_sd(sdA, semA)
        load_dv(jnp.minimum(t0 + 2, last), dvCA, semDA)
        cur = compact_chunk(t0, sdA, cur)
        wait_dv(dvCA, semDA)
        issue_sd(dvCA, sdA, semA)
        wait_sd(sdB, semB)
        load_dv(jnp.minimum(t0 + 3, last), dvCB, semDB)
        cur = compact_chunk(t0 + 1, sdB, cur)
        return cur

    cursor = lax.fori_loop(0, (CHUNKS_PER_SUB - 1) // 2, pairA,
                           jnp.int32(0))
    wait_dv(dvCB, semDB)
    wait_sd(sdA, semA)
    cursor = compact_chunk(last, sdA, cursor)

    for half in range(2):
        for k in range(CHUNK // LANES):
            padidx[pl.ds(k * LANES, LANES)] = (
                list_base + cursor + half * CHUNK + k * LANES + iota16)
        pltpu.sync_copy(padpk, cl_sh.at[padidx])

    nc0 = (cursor + CHUNK - 1) // CHUNK
    nchunks = nc0 + (nc0 % 2)

    def copy_idx(t, svc, pvc):
        pltpu.sync_copy(cl_sh.at[pl.ds(list_base + t * CHUNK, CHUNK)], pkC)
        for k in range(CHUNK // LANES):
            sl = pl.ds(k * LANES, LANES)
            v = pkC[sl]
            svc[sl] = v & 0xFFFF
            pvc[sl] = lax.shift_right_logical(v, 16)

    def issue_rows(svc, buf, sem):
        pltpu.async_copy(x_hbm.at[svc], buf, sem)

    def wait_rows(buf, sem):
        pltpu.make_async_copy(x_hbm.at[svcA], buf, sem).wait()

    def scatter_chunk(xbuf, pvc):
        pltpu.sync_copy(xbuf, acc_sh.at[pvc], add=True)
        pltpu.sync_copy(ones, cnt_sh.at[pvc], add=True)

    copy_idx(0, svcA, pvcA)
    issue_rows(svcA, xrowsA, semA)

    def pairB(i, carry):
        t0 = 2 * i
        copy_idx(t0 + 1, svcB, pvcB)
        issue_rows(svcB, xrowsB, semB)
        wait_rows(xrowsA, semA)
        scatter_chunk(xrowsA, pvcA)
        copy_idx(jnp.minimum(t0 + 2, nchunks - 1), svcA, pvcA)
        issue_rows(svcA, xrowsA, semA)
        wait_rows(xrowsB, semB)
        scatter_chunk(xrowsB, pvcB)
        return carry

    lax.fori_loop(0, nchunks // 2, pairB, 0)
    wait_rows(xrowsA, semA)

    plsc.subcore_barrier()

    pltpu.sync_copy(slot_sh.at[nv], g)
    for r in range(R):
        for k in range(NODES_PER_SUB // LANES):
            sl = pl.ds(k * LANES, LANES)
            gp[sl] = g[sl] * R + r
        pltpu.sync_copy(acc_sh.at[gp], xrowsA)
        fbase = (c * R + r) * NODES + nbase
        pltpu.sync_copy(xrowsA, acc2_hbm.at[pl.ds(fbase, NODES_PER_SUB)])
        pltpu.sync_copy(cnt_sh.at[gp], cv)
        pltpu.sync_copy(cv, cnt2_hbm.at[pl.ds(fbase, NODES_PER_SUB)])

    xb = wid * XN_PER_W
    pltpu.sync_copy(nodes_hbm.at[pl.ds(xb, XN_PER_W)], nv2)
    pltpu.async_copy(x_hbm.at[nv2], xrowsB.at[pl.ds(0, XN_PER_W)],
                     semA).wait()
    pltpu.sync_copy(xrowsB.at[pl.ds(0, XN_PER_W)],
                    xn_hbm.at[pl.ds(xb, XN_PER_W)])


_SC_SCRATCH = [
    pltpu.VMEM_SHARED((SLOT_PAD,), jnp.int32),
    pltpu.VMEM_SHARED((ACC_ROWS, D), jnp.float32),
    pltpu.VMEM_SHARED((ACC_ROWS,), jnp.float32),
    pltpu.VMEM_SHARED((NS * LIST_CAP,), jnp.int32),
    pltpu.VMEM((EDGES_PER_SUB,), jnp.int32),
    pltpu.VMEM((CHUNK,), jnp.int32),
    pltpu.VMEM((CHUNK,), jnp.int32),
    pltpu.VMEM((CHUNK,), jnp.int32),
    pltpu.VMEM((CHUNK,), jnp.int32),
    pltpu.VMEM((CHUNK,), jnp.int32),
    pltpu.VMEM((CHUNK,), jnp.int32),
    pltpu.VMEM((CHUNK,), jnp.int32),
    pltpu.VMEM((CHUNK,), jnp.int32),
    pltpu.VMEM((CHUNK,), jnp.int32),
    pltpu.VMEM((CHUNK,), jnp.int32),
    pltpu.VMEM((CHUNK,), jnp.int32),
    pltpu.VMEM((CHUNK, D), jnp.float32),
    pltpu.VMEM((CHUNK, D), jnp.float32),
    pltpu.VMEM((CHUNK,), jnp.int32),
    pltpu.VMEM((CHUNK,), jnp.int32),
    pltpu.VMEM((SLOT_PER_SUB,), jnp.float32),
    pltpu.VMEM((SLOT_PER_SUB,), jnp.int32),
    pltpu.VMEM((CHUNK,), jnp.float32),
    pltpu.VMEM((NODES_PER_SUB,), jnp.int32),
    pltpu.VMEM((NODES_PER_SUB,), jnp.int32),
    pltpu.VMEM((NODES_PER_SUB,), jnp.int32),
    pltpu.VMEM((NODES_PER_SUB,), jnp.int32),
    pltpu.VMEM((NODES_PER_SUB,), jnp.float32),
    pltpu.VMEM((XN_PER_W,), jnp.int32),
    pltpu.SemaphoreType.DMA,
    pltpu.SemaphoreType.DMA,
    pltpu.SemaphoreType.DMA,
    pltpu.SemaphoreType.DMA,
]

_SC_OUT = (
    jax.ShapeDtypeStruct((NC * R * NODES, D), jnp.float32),
    jax.ShapeDtypeStruct((NC * R * NODES,), jnp.float32),
    jax.ShapeDtypeStruct((NODES, D), jnp.float32),
)

_sc_call_cached = None


def _sc_call(*args):
    global _sc_call_cached
    if _sc_call_cached is None:
        _sc_call_cached = pl.kernel(
            _sc_body,
            out_type=_SC_OUT,
            mesh=plsc.VectorSubcoreMesh(core_axis_name="c",
                                        subcore_axis_name="s",
                                        num_cores=NC, num_subcores=NS),
            scratch_types=_SC_SCRATCH,
        )
    return _sc_call_cached(*args)


def _tc_body(acc2_ref, cnt2_ref, xn_ref, wrel_ref, wroot_ref, brg_ref,
             wfc_ref, bfc_ref, out_ref):
    agg = jnp.zeros((NODES, D), jnp.float32)
    for r in range(R):
        acc_r = acc2_ref[0, r] + acc2_ref[1, r]
        cnt_r = cnt2_ref[0, r] + cnt2_ref[1, r]
        norm = 1.0 / jnp.maximum(cnt_r, 1.0)
        agg = agg + jnp.dot(acc_r * norm, wrel_ref[r],
                            preferred_element_type=jnp.float32)
    h = agg + jnp.dot(xn_ref[...], wroot_ref[...],
                      preferred_element_type=jnp.float32) + brg_ref[...]
    h = jnp.maximum(h, 0.0)
    comb = jnp.concatenate([h[:B], h[B:]], axis=1)
    out = jnp.dot(comb, wfc_ref[...],
                  preferred_element_type=jnp.float32) + bfc_ref[...]
    out_ref[...] = jnp.maximum(out, 0.0)


def _tc_call(acc2, cnt2, xn, W_rel, W_root, brg, W_fc, bfc):
    return pl.pallas_call(
        _tc_body,
        out_shape=jax.ShapeDtypeStruct((B, H), jnp.float32),
    )(acc2, cnt2, xn, W_rel, W_root, brg, W_fc, bfc)


def kernel(x, edge_index, edge_type, nest_tensor, food_tensor,
           W_rel, W_root, b_rgcn, W_fc, b_fc):
    src = edge_index[0].astype(jnp.int32)
    dst = edge_index[1].astype(jnp.int32)
    et = edge_type.astype(jnp.int32)
    nodes = jnp.concatenate([nest_tensor, food_tensor]).astype(jnp.int32)
    pad = E_PAD - E
    epk = src | (et << 16)
    epk = jnp.concatenate([epk, jnp.zeros((pad,), jnp.int32)])
    dst = jnp.concatenate([dst, jnp.full((pad,), N, jnp.int32)])

    acc2, cnt2, xn = _sc_call(x, epk, dst, nodes)
    acc2 = acc2.reshape(NC, R, NODES, D)
    cnt2 = cnt2.reshape(NC, R, NODES, 1)
    return _tc_call(acc2, cnt2, xn, W_rel, W_root,
                    b_rgcn.reshape(1, D), W_fc, b_fc.reshape(1, H))


# speedup vs baseline: 17.7654x; 1.0002x over previous
import jax
import jax.numpy as jnp
from jax import lax
from jax.experimental import pallas as pl
from jax.experimental.pallas import tpu as pltpu
from jax.experimental.pallas import tpu_sc as plsc

N = 10000
E = 320000
D = 128
R = 4
H = 256
B = 1024

NC = 2
NS = 16
LANES = 16

NODES = 2 * B
SENTINEL = NODES
SLOT_PAD = 10240
ACC_ROWS = 8320
ROWS_PER_SUB = ACC_ROWS // NS
SLOT_PER_SUB = SLOT_PAD // NS
CHUNK = 128
CHUNKS_PER_SUB = 79
EDGES_PER_SUB = CHUNKS_PER_SUB * CHUNK
E_PAD = NC * NS * EDGES_PER_SUB
NODES_PER_SUB = NODES // NS
XN_PER_W = NODES // (NC * NS)
LIST = EDGES_PER_SUB + 2 * CHUNK
TRASH = LIST
LIST_CAP = LIST + LANES
DUMMY_PAIR = SENTINEL * R
PAD_PACK = DUMMY_PAIR << 16


def _sc_body(x_hbm, epk_hbm, dst_hbm, nodes_hbm,
             acc2_hbm, cnt2_hbm, xn_hbm,
             slot_sh, acc_sh, cnt_sh, cl_sh,
             epk_all, dvCA, dvCB, sdA, sdB, pkbuf, posv,
             pkC, svcA, svcB, pvcA, pvcB,
             xrowsA, xrowsB, padidx, padpk,
             zline, sbuf, ones, nv, vals, g, gp, cv, nv2,
             semA, semB, semDA, semDB):
    c = lax.axis_index("c")
    s = lax.axis_index("s")
    wid = s * NC + c
    list_base = s * LIST_CAP

    zero16 = jnp.zeros((LANES,), jnp.float32)
    iota16 = lax.iota(jnp.int32, LANES)
    sent16 = jnp.full((LANES,), SENTINEL, jnp.int32)
    one16 = jnp.ones((LANES,), jnp.float32)

    ebase = wid * EDGES_PER_SUB
    pltpu.async_copy(epk_hbm.at[pl.ds(ebase, EDGES_PER_SUB)], epk_all, semDA)

    scope_init = jax.named_scope("ph_init")
    scope_init.__enter__()

    def zrow_body(i, carry):
        for k in range(D // LANES):
            xrowsA[i, pl.ds(k * LANES, LANES)] = zero16
        return carry

    lax.fori_loop(0, CHUNK, zrow_body, 0)
    for k in range(SLOT_PER_SUB // LANES):
        zline[pl.ds(k * LANES, LANES)] = zero16
        sbuf[pl.ds(k * LANES, LANES)] = sent16
    for k in range(CHUNK // LANES):
        ones[pl.ds(k * LANES, LANES)] = one16
        padpk[pl.ds(k * LANES, LANES)] = jnp.full((LANES,), PAD_PACK,
                                                  jnp.int32)

    base_r = s * ROWS_PER_SUB
    for j in range(4):
        pltpu.sync_copy(xrowsA, acc_sh.at[pl.ds(base_r + j * CHUNK, CHUNK)])
    pltpu.sync_copy(xrowsA.at[pl.ds(0, 8)], acc_sh.at[pl.ds(base_r + 512, 8)])
    pltpu.sync_copy(zline.at[pl.ds(0, ROWS_PER_SUB)],
                    cnt_sh.at[pl.ds(base_r, ROWS_PER_SUB)])
    pltpu.sync_copy(sbuf, slot_sh.at[pl.ds(s * SLOT_PER_SUB, SLOT_PER_SUB)])

    plsc.subcore_barrier()
    scope_init.__exit__(None, None, None)
    scope_slot = jax.named_scope("ph_slot")
    scope_slot.__enter__()

    nbase = s * NODES_PER_SUB
    pltpu.sync_copy(nodes_hbm.at[pl.ds(nbase, NODES_PER_SUB)], nv)
    for k in range(NODES_PER_SUB // LANES):
        vals[pl.ds(k * LANES, LANES)] = nbase + k * LANES + iota16
    pltpu.sync_copy(vals, slot_sh.at[nv])

    plsc.subcore_barrier()
    scope_slot.__exit__(None, None, None)
    scope_pa = jax.named_scope("ph_compact")
    scope_pa.__enter__()

    pltpu.make_async_copy(epk_hbm.at[pl.ds(0, EDGES_PER_SUB)], epk_all,
                          semDA).wait()

    def load_dv(t, buf, sem):
        pltpu.async_copy(dst_hbm.at[pl.ds(ebase + t * CHUNK, CHUNK)],
                         buf, sem)

    def wait_dv(buf, sem):
        pltpu.make_async_copy(dst_hbm.at[pl.ds(0, CHUNK)], buf, sem).wait()

    def issue_sd(dvC, buf, sem):
        pltpu.async_copy(slot_sh.at[dvC], buf, sem)

    def wait_sd(buf, sem):
        pltpu.make_async_copy(slot_sh.at[dvCA], buf, sem).wait()

    def compact_chunk(t, sdref, cur):
        for k in range(CHUNK // LANES):
            sl = pl.ds(k * LANES, LANES)
            off = t * CHUNK + k * LANES
            sd16 = sdref[sl]
            ep16 = epk_all[pl.ds(off, LANES)]
            ev16 = lax.shift_right_logical(ep16, 16)
            sv16 = ep16 & 0xFFFF
            valid = sd16 < SENTINEL
            pv16 = sd16 * R + ev16
            csum = jnp.where(valid, 1, 0).astype(jnp.int32)
            for sh in (1, 2, 4, 8):
                idx = jnp.maximum(iota16 - sh, 0)
                shifted = jnp.take(csum, idx, mode="wrap")
                csum = csum + jnp.where(iota16 >= sh, shifted, 0)
            pos16 = jnp.where(valid, list_base + cur + csum - 1,
                              list_base + TRASH + iota16)
            pkbuf[sl] = sv16 | lax.shift_left(pv16, 16)
            posv[sl] = pos16
            cur = cur + csum[LANES - 1]
        pltpu.sync_copy(pkbuf, cl_sh.at[posv])
        return cur

    last = CHUNKS_PER_SUB - 1
    load_dv(0, dvCA, semDA)
    wait_dv(dvCA, semDA)
    issue_sd(dvCA, sdA, semA)
    load_dv(1, dvCB, semDB)

    def pairA(i, cur):
        t0 = 2 * i
        wait_dv(dvCB, semDB)
        issue_sd(dvCB, sdB, semB)
        wait_sd(sdA, semA)
        load_dv(jnp.minimum(t0 + 2, last), dvCA, semDA)
        cur = compact_chunk(t0, sdA, cur)
        wait_dv(dvCA, semDA)
        issue_sd(dvCA, sdA, semA)
        wait_sd(sdB, semB)
        load_dv(jnp.minimum(t0 + 3, last), dvCB, semDB)
        cur = compact_chunk(t0 + 1, sdB, cur)
        return cur

    cursor = lax.fori_loop(0, (CHUNKS_PER_SUB - 1) // 2, pairA,
                           jnp.int32(0))
    wait_dv(dvCB, semDB)
    wait_sd(sdA, semA)
    cursor = compact_chunk(last, sdA, cursor)

    for half in range(2):
        for k in range(CHUNK // LANES):
            padidx[pl.ds(k * LANES, LANES)] = (
                list_base + cursor + half * CHUNK + k * LANES + iota16)
        pltpu.sync_copy(padpk, cl_sh.at[padidx])

    nc0 = (cursor + CHUNK - 1) // CHUNK
    nchunks = nc0 + (nc0 % 2)
    scope_pa.__exit__(None, None, None)
    scope_pb = jax.named_scope("ph_rows")
    scope_pb.__enter__()

    def copy_idx(t, svc, pvc):
        pltpu.sync_copy(cl_sh.at[pl.ds(list_base + t * CHUNK, CHUNK)], pkC)
        for k in range(CHUNK // LANES):
            sl = pl.ds(k * LANES, LANES)
            v = pkC[sl]
            svc[sl] = v & 0xFFFF
            pvc[sl] = lax.shift_right_logical(v, 16)

    def issue_rows(svc, buf, sem):
        pltpu.async_copy(x_hbm.at[svc], buf, sem)

    def wait_rows(buf, sem):
        pltpu.make_async_copy(x_hbm.at[svcA], buf, sem).wait()

    def scatter_chunk(xbuf, pvc):
        pltpu.sync_copy(xbuf, acc_sh.at[pvc], add=True)
        pltpu.sync_copy(ones, cnt_sh.at[pvc], add=True)

    copy_idx(0, svcA, pvcA)
    issue_rows(svcA, xrowsA, semA)

    def pairB(i, carry):
        t0 = 2 * i
        copy_idx(t0 + 1, svcB, pvcB)
        issue_rows(svcB, xrowsB, semB)
        wait_rows(xrowsA, semA)
        scatter_chunk(xrowsA, pvcA)
        copy_idx(jnp.minimum(t0 + 2, nchunks - 1), svcA, pvcA)
        issue_rows(svcA, xrowsA, semA)
        wait_rows(xrowsB, semB)
        scatter_chunk(xrowsB, pvcB)
        return carry

    lax.fori_loop(0, nchunks // 2, pairB, 0)
    wait_rows(xrowsA, semA)

    plsc.subcore_barrier()
    scope_pb.__exit__(None, None, None)
    scope_rd = jax.named_scope("ph_redist")
    scope_rd.__enter__()

    pltpu.sync_copy(slot_sh.at[nv], g)
    for r in range(R):
        for k in range(NODES_PER_SUB // LANES):
            sl = pl.ds(k * LANES, LANES)
            gp[sl] = g[sl] * R + r
        pltpu.sync_copy(acc_sh.at[gp], xrowsA)
        fbase = (c * R + r) * NODES + nbase
        pltpu.sync_copy(xrowsA, acc2_hbm.at[pl.ds(fbase, NODES_PER_SUB)])
        pltpu.sync_copy(cnt_sh.at[gp], cv)
        pltpu.sync_copy(cv, cnt2_hbm.at[pl.ds(fbase, NODES_PER_SUB)])

    xb = wid * XN_PER_W
    pltpu.sync_copy(nodes_hbm.at[pl.ds(xb, XN_PER_W)], nv2)
    pltpu.async_copy(x_hbm.at[nv2], xrowsB.at[pl.ds(0, XN_PER_W)],
                     semA).wait()
    pltpu.sync_copy(xrowsB.at[pl.ds(0, XN_PER_W)],
                    xn_hbm.at[pl.ds(xb, XN_PER_W)])
    scope_rd.__exit__(None, None, None)


_SC_SCRATCH = [
    pltpu.VMEM_SHARED((SLOT_PAD,), jnp.int32),
    pltpu.VMEM_SHARED((ACC_ROWS, D), jnp.float32),
    pltpu.VMEM_SHARED((ACC_ROWS,), jnp.float32),
    pltpu.VMEM_SHARED((NS * LIST_CAP,), jnp.int32),
    pltpu.VMEM((EDGES_PER_SUB,), jnp.int32),
    pltpu.VMEM((CHUNK,), jnp.int32),
    pltpu.VMEM((CHUNK,), jnp.int32),
    pltpu.VMEM((CHUNK,), jnp.int32),
    pltpu.VMEM((CHUNK,), jnp.int32),
    pltpu.VMEM((CHUNK,), jnp.int32),
    pltpu.VMEM((CHUNK,), jnp.int32),
    pltpu.VMEM((CHUNK,), jnp.int32),
    pltpu.VMEM((CHUNK,), jnp.int32),
    pltpu.VMEM((CHUNK,), jnp.int32),
    pltpu.VMEM((CHUNK,), jnp.int32),
    pltpu.VMEM((CHUNK,), jnp.int32),
    pltpu.VMEM((CHUNK, D), jnp.float32),
    pltpu.VMEM((CHUNK, D), jnp.float32),
    pltpu.VMEM((CHUNK,), jnp.int32),
    pltpu.VMEM((CHUNK,), jnp.int32),
    pltpu.VMEM((SLOT_PER_SUB,), jnp.float32),
    pltpu.VMEM((SLOT_PER_SUB,), jnp.int32),
    pltpu.VMEM((CHUNK,), jnp.float32),
    pltpu.VMEM((NODES_PER_SUB,), jnp.int32),
    pltpu.VMEM((NODES_PER_SUB,), jnp.int32),
    pltpu.VMEM((NODES_PER_SUB,), jnp.int32),
    pltpu.VMEM((NODES_PER_SUB,), jnp.int32),
    pltpu.VMEM((NODES_PER_SUB,), jnp.float32),
    pltpu.VMEM((XN_PER_W,), jnp.int32),
    pltpu.SemaphoreType.DMA,
    pltpu.SemaphoreType.DMA,
    pltpu.SemaphoreType.DMA,
    pltpu.SemaphoreType.DMA,
]

_SC_OUT = (
    jax.ShapeDtypeStruct((NC * R * NODES, D), jnp.float32),
    jax.ShapeDtypeStruct((NC * R * NODES,), jnp.float32),
    jax.ShapeDtypeStruct((NODES, D), jnp.float32),
)

_sc_call_cached = None


def _sc_call(*args):
    global _sc_call_cached
    if _sc_call_cached is None:
        _sc_call_cached = pl.kernel(
            _sc_body,
            out_type=_SC_OUT,
            mesh=plsc.VectorSubcoreMesh(core_axis_name="c",
                                        subcore_axis_name="s",
                                        num_cores=NC, num_subcores=NS),
            scratch_types=_SC_SCRATCH,
        )
    return _sc_call_cached(*args)


def _tc_body(acc2_ref, cnt2_ref, xn_ref, wrel_ref, wroot_ref, brg_ref,
             wfc_ref, bfc_ref, out_ref):
    agg = jnp.zeros((NODES, D), jnp.float32)
    for r in range(R):
        acc_r = acc2_ref[0, r] + acc2_ref[1, r]
        cnt_r = cnt2_ref[0, r] + cnt2_ref[1, r]
        norm = 1.0 / jnp.maximum(cnt_r, 1.0)
        agg = agg + jnp.dot(acc_r * norm, wrel_ref[r],
                            preferred_element_type=jnp.float32)
    h = agg + jnp.dot(xn_ref[...], wroot_ref[...],
                      preferred_element_type=jnp.float32) + brg_ref[...]
    h = jnp.maximum(h, 0.0)
    comb = jnp.concatenate([h[:B], h[B:]], axis=1)
    out = jnp.dot(comb, wfc_ref[...],
                  preferred_element_type=jnp.float32) + bfc_ref[...]
    out_ref[...] = jnp.maximum(out, 0.0)


def _tc_call(acc2, cnt2, xn, W_rel, W_root, brg, W_fc, bfc):
    return pl.pallas_call(
        _tc_body,
        out_shape=jax.ShapeDtypeStruct((B, H), jnp.float32),
    )(acc2, cnt2, xn, W_rel, W_root, brg, W_fc, bfc)


def kernel(x, edge_index, edge_type, nest_tensor, food_tensor,
           W_rel, W_root, b_rgcn, W_fc, b_fc):
    src = edge_index[0].astype(jnp.int32)
    dst = edge_index[1].astype(jnp.int32)
    et = edge_type.astype(jnp.int32)
    nodes = jnp.concatenate([nest_tensor, food_tensor]).astype(jnp.int32)
    pad = E_PAD - E
    epk = src | (et << 16)
    epk = jnp.concatenate([epk, jnp.zeros((pad,), jnp.int32)])
    dst = jnp.concatenate([dst, jnp.full((pad,), N, jnp.int32)])

    acc2, cnt2, xn = _sc_call(x, epk, dst, nodes)
    acc2 = acc2.reshape(NC, R, NODES, D)
    cnt2 = cnt2.reshape(NC, R, NODES, 1)
    return _tc_call(acc2, cnt2, xn, W_rel, W_root,
                    b_rgcn.reshape(1, D), W_fc, b_fc.reshape(1, H))
